# Initial kernel scaffold; baseline (speedup 1.0000x reference)
import jax
import jax.numpy as jnp
from jax.experimental import pallas as pl


def kernel(atom_features, bond_features, edge_index, line_edge_index, graph_ids, W_i, W_h, W_o_w, W_o_b):
    raise NotImplementedError("write your pallas kernel here")


# speedup vs baseline: 2.5642x; 2.5642x over previous
import functools

import jax
import jax.numpy as jnp
from jax import lax
from jax.experimental import pallas as pl
from jax.experimental.pallas import tpu as pltpu
from jax.experimental.pallas import tpu_sc as plsc

H = 128
NC = 2
NS = 16
LANES = 16


def _mm_body(x_ref, w_ref, o_ref):
    o_ref[...] = jnp.dot(x_ref[...], w_ref[...],
                         preferred_element_type=jnp.float32)


def _matmul(x, w_t, block):
    m, k = x.shape
    return pl.pallas_call(
        _mm_body,
        grid=(m // block,),
        in_specs=[pl.BlockSpec((block, k), lambda i: (i, 0)),
                  pl.BlockSpec((k, H), lambda i: (0, 0))],
        out_specs=pl.BlockSpec((block, H), lambda i: (i, 0)),
        out_shape=jax.ShapeDtypeStruct((m, H), jnp.float32),
    )(x, w_t)


def _add_relu_body(g_ref, b_ref, mi_ref, msg_ref):
    s = g_ref[...] + b_ref[...]
    mi_ref[...] = s
    msg_ref[...] = jnp.maximum(s, 0.0)


def _add_relu(g, b, block):
    m = g.shape[0]
    return pl.pallas_call(
        _add_relu_body,
        grid=(m // block,),
        in_specs=[pl.BlockSpec((block, H), lambda i: (i, 0)),
                  pl.BlockSpec((block, H), lambda i: (i, 0))],
        out_specs=[pl.BlockSpec((block, H), lambda i: (i, 0)),
                   pl.BlockSpec((block, H), lambda i: (i, 0))],
        out_shape=[jax.ShapeDtypeStruct((m, H), jnp.float32),
                   jax.ShapeDtypeStruct((m, H), jnp.float32)],
    )(g, b)


def _round_body(a_ref, w_ref, mi_ref, o_ref):
    o_ref[...] = jnp.maximum(
        mi_ref[...] + jnp.dot(a_ref[...], w_ref[...],
                              preferred_element_type=jnp.float32), 0.0)


def _round_matmul(accum, w_h_t, msg_input, block):
    m = accum.shape[0]
    return pl.pallas_call(
        _round_body,
        grid=(m // block,),
        in_specs=[pl.BlockSpec((block, H), lambda i: (i, 0)),
                  pl.BlockSpec((H, H), lambda i: (0, 0)),
                  pl.BlockSpec((block, H), lambda i: (i, 0))],
        out_specs=pl.BlockSpec((block, H), lambda i: (i, 0)),
        out_shape=jax.ShapeDtypeStruct((m, H), jnp.float32),
    )(accum, w_h_t, msg_input)


def _final_body(atom_ref, m_ref, gid_ref, w1_ref, w2_ref, b_ref, o_ref,
                sums_ref, cnt_ref, *, nblocks, block, n_graphs):
    i = pl.program_id(0)

    @pl.when(i == 0)
    def _():
        sums_ref[...] = jnp.zeros_like(sums_ref)
        cnt_ref[...] = jnp.zeros_like(cnt_ref)

    h = jnp.maximum(
        jnp.dot(atom_ref[...], w1_ref[...], preferred_element_type=jnp.float32)
        + jnp.dot(m_ref[...], w2_ref[...], preferred_element_type=jnp.float32)
        + b_ref[...], 0.0)
    gids = gid_ref[...].reshape(block, 1)
    onehot = (gids == lax.broadcasted_iota(jnp.int32, (block, n_graphs), 1)
              ).astype(jnp.float32)
    sums_ref[...] += lax.dot_general(
        onehot, h, (((0,), (0,)), ((), ())),
        preferred_element_type=jnp.float32)
    cnt_ref[...] += jnp.sum(onehot, axis=0, keepdims=True)

    @pl.when(i == nblocks - 1)
    def _():
        counts = jnp.maximum(cnt_ref[...], 1.0).reshape(n_graphs, 1)
        o_ref[...] = sums_ref[...] / counts


def _final_readout(atom_features, m, graph_ids, w1_t, w2_t, bias, n_graphs,
                   block):
    n, fdim = atom_features.shape
    nblocks = n // block
    gid3 = graph_ids.reshape(nblocks, 1, block)
    body = functools.partial(_final_body, nblocks=nblocks, block=block,
                             n_graphs=n_graphs)
    return pl.pallas_call(
        body,
        grid=(nblocks,),
        in_specs=[pl.BlockSpec((block, fdim), lambda i: (i, 0)),
                  pl.BlockSpec((block, H), lambda i: (i, 0)),
                  pl.BlockSpec((1, 1, block), lambda i: (i, 0, 0)),
                  pl.BlockSpec((fdim, H), lambda i: (0, 0)),
                  pl.BlockSpec((H, H), lambda i: (0, 0)),
                  pl.BlockSpec((1, H), lambda i: (0, 0))],
        out_specs=pl.BlockSpec((n_graphs, H), lambda i: (0, 0)),
        out_shape=jax.ShapeDtypeStruct((n_graphs, H), jnp.float32),
        scratch_shapes=[pltpu.VMEM((n_graphs, H), jnp.float32),
                        pltpu.VMEM((1, n_graphs), jnp.float32)],
    )(atom_features, m, gid3, w1_t, w2_t, bias)


def _sc_gather(table, idx):
    e = idx.shape[0]
    per_w = e // (NC * NS)
    ch = 200
    assert per_w % ch == 0 and ch % 8 == 0
    mesh = plsc.VectorSubcoreMesh(core_axis_name="c", subcore_axis_name="s")

    @functools.partial(
        pl.kernel,
        out_type=jax.ShapeDtypeStruct((e, H), jnp.float32),
        mesh=mesh,
        compiler_params=pltpu.CompilerParams(needs_layout_passes=False),
        scratch_types=[pltpu.VMEM((ch,), jnp.int32),
                       pltpu.VMEM((ch, H), jnp.float32),
                       pltpu.SemaphoreType.DMA],
    )
    def k(table_hbm, idx_hbm, out_hbm, idx_v, rows_v, sem):
        wid = lax.axis_index("s") * NC + lax.axis_index("c")
        base = wid * per_w

        def body(j, carry):
            o = base + j * ch
            pltpu.sync_copy(idx_hbm.at[pl.ds(o, ch)], idx_v)
            pltpu.async_copy(table_hbm.at[idx_v], rows_v, sem).wait()
            pltpu.sync_copy(rows_v, out_hbm.at[pl.ds(o, ch)])
            return carry

        lax.fori_loop(0, per_w // ch, body, 0)

    return k(table, idx)


def _sc_segment_sum(data, gsrc, dst, out_rows, wr):
    j_tot = gsrc.shape[0]
    assert j_tot % NS == 0
    strip = j_tot // NS
    rp = -(-out_rows // (NS * 8)) * (NS * 8)
    nwin = -(-rp // wr)
    tail = rp - (nwin - 1) * wr
    br = 128
    ch2 = 2000
    cap = 2304
    assert strip % ch2 == 0 and ch2 % LANES == 0 and cap % br == 0
    assert wr % (NS * 8) == 0 and tail % (NS * 8) == 0
    zb = 64
    mesh = plsc.VectorSubcoreMesh(core_axis_name="c", subcore_axis_name="s")

    @functools.partial(
        pl.kernel,
        out_type=jax.ShapeDtypeStruct((rp, H), jnp.float32),
        mesh=mesh,
        compiler_params=pltpu.CompilerParams(needs_layout_passes=False),
        scratch_types=[
            pltpu.VMEM_SHARED((wr + LANES, H), jnp.float32),
            pltpu.VMEM((ch2,), jnp.int32),
            pltpu.VMEM((ch2,), jnp.int32),
            pltpu.VMEM((cap,), jnp.int32),
            pltpu.VMEM((cap // br, br), jnp.int32),
            pltpu.VMEM((br, H), jnp.float32),
            pltpu.VMEM((zb, H), jnp.float32),
            pltpu.SemaphoreType.DMA,
        ],
    )
    def k(data_hbm, gsrc_hbm, dst_hbm, out_hbm, acc_sh, in_dst, in_gsrc,
          cg, cs, rows_v, zbuf, sem):
        cid = lax.axis_index("c")
        tid = lax.axis_index("s")

        def zrow(q, carry):
            r = q // (H // LANES)
            l = q % (H // LANES)
            zbuf[r, pl.ds(l * LANES, LANES)] = jnp.zeros((LANES,),
                                                         jnp.float32)
            return carry

        lax.fori_loop(0, zb * (H // LANES), zrow, 0)

        def fire(b):
            pltpu.async_copy(data_hbm.at[cg.at[pl.ds(b * br, br)]],
                             rows_v, sem).wait()
            pltpu.sync_copy(rows_v, acc_sh.at[cs.at[b]], add=True)

        def process_window(w, wrows, share):
            base = w * wr
            for z in range(wr // NS // zb):
                pltpu.sync_copy(
                    zbuf, acc_sh.at[pl.ds(tid * (wr // NS) + z * zb, zb)])
            plsc.subcore_barrier()

            sbase = tid * strip

            def chunk(ci, cnt):
                off = sbase + ci * ch2
                pltpu.sync_copy(dst_hbm.at[pl.ds(off, ch2)], in_dst)
                pltpu.sync_copy(gsrc_hbm.at[pl.ds(off, ch2)], in_gsrc)

                def vreg(vi, cnt):
                    iota16 = lax.iota(jnp.int32, LANES)
                    d = in_dst[pl.ds(vi * LANES, LANES)]
                    g = in_gsrc[pl.ds(vi * LANES, LANES)]
                    msk = (d >= base) & (d < base + wrows)
                    pref = plsc.cumsum(msk.astype(jnp.int32))
                    pos = jnp.where(msk, cnt + pref - 1,
                                    cap - LANES + iota16)
                    plsc.store_scatter(cg, [pos], g)
                    plsc.store_scatter(cs, [pos // br, pos % br], d - base)
                    return cnt + pref[LANES - 1]

                cnt = lax.fori_loop(0, ch2 // LANES, vreg, cnt)

                nb = cnt // br

                def b_loop(b, carry):
                    fire(b)
                    return carry

                lax.fori_loop(0, nb, b_loop, 0)

                for q in range(br // LANES):
                    vg = cg[pl.ds(nb * br + q * LANES, LANES)]
                    cg[pl.ds(q * LANES, LANES)] = vg
                    vs = cs[nb, pl.ds(q * LANES, LANES)]
                    cs[0, pl.ds(q * LANES, LANES)] = vs
                return cnt - nb * br

            cnt = lax.fori_loop(0, strip // ch2, chunk, jnp.int32(0))

            iota16 = lax.iota(jnp.int32, LANES)
            for p in range(br // LANES):
                pp = cnt + p * LANES + iota16
                plsc.store_scatter(cg, [pp], iota16)
                plsc.store_scatter(cs, [pp // br, pp % br], wr + iota16)

            @pl.when(cnt > 0)
            def _():
                fire(0)

            plsc.subcore_barrier()

            pltpu.sync_copy(acc_sh.at[pl.ds(tid * share, share)],
                            out_hbm.at[pl.ds(base + tid * share, share)])

        nfull = (nwin - 1 - cid + NC - 1) // NC

        def wloop(q, carry):
            process_window(q * NC + cid, wr, wr // NS)
            return carry

        lax.fori_loop(0, nfull, wloop, 0)

        @pl.when(cid == (nwin - 1) % NC)
        def _():
            process_window(nwin - 1, tail, tail // NS)

    res = k(data, gsrc, dst)
    return res if rp == out_rows else res[:out_rows]


def kernel(atom_features, bond_features, edge_index, line_edge_index,
           graph_ids, W_i, W_h, W_o_w, W_o_b):
    n, atom_fdim = atom_features.shape
    e = bond_features.shape[0]
    n_graphs = 64
    depth = 3

    src, dst = edge_index[0], edge_index[1]
    lsrc, ldst = line_edge_index[0], line_edge_index[1]

    w_i_a_t = W_i[:, :atom_fdim].T
    w_i_b_t = W_i[:, atom_fdim:].T
    w_h_t = W_h.T
    w_o_a_t = W_o_w[:, :atom_fdim].T
    w_o_m_t = W_o_w[:, atom_fdim:].T
    bias = W_o_b.reshape(1, H)

    a_proj = _matmul(atom_features, w_i_a_t, block=1000)
    b_proj = _matmul(bond_features, w_i_b_t, block=2000)
    g_rows = _sc_gather(a_proj, src)
    msg_input, msg = _add_relu(g_rows, b_proj, block=2000)

    for _ in range(depth - 1):
        accum = _sc_segment_sum(msg, lsrc, ldst, e, wr=10240)
        msg = _round_matmul(accum, w_h_t, msg_input, block=2000)

    edge_iota = jnp.arange(e, dtype=jnp.int32)
    m = _sc_segment_sum(msg, edge_iota, dst, n, wr=5120)

    return _final_readout(atom_features, m, graph_ids, w_o_a_t, w_o_m_t,
                          bias, n_graphs, block=1000)


# speedup vs baseline: 3.6849x; 1.4371x over previous
import functools

import jax
import jax.numpy as jnp
from jax import lax
from jax.experimental import pallas as pl
from jax.experimental.pallas import tpu as pltpu
from jax.experimental.pallas import tpu_sc as plsc

H = 128
NC = 2
NS = 16
LANES = 16


def _mm_body(x_ref, w_ref, o_ref):
    o_ref[...] = jnp.dot(x_ref[...], w_ref[...],
                         preferred_element_type=jnp.float32)


def _matmul(x, w_t, block):
    m, k = x.shape
    return pl.pallas_call(
        _mm_body,
        grid=(m // block,),
        in_specs=[pl.BlockSpec((block, k), lambda i: (i, 0)),
                  pl.BlockSpec((k, H), lambda i: (0, 0))],
        out_specs=pl.BlockSpec((block, H), lambda i: (i, 0)),
        out_shape=jax.ShapeDtypeStruct((m, H), jnp.float32),
    )(x, w_t)


def _add_relu_body(g_ref, b_ref, mi_ref, msg_ref):
    s = g_ref[...] + b_ref[...]
    mi_ref[...] = s
    msg_ref[...] = jnp.maximum(s, 0.0)


def _add_relu(g, b, block):
    m = g.shape[0]
    return pl.pallas_call(
        _add_relu_body,
        grid=(m // block,),
        in_specs=[pl.BlockSpec((block, H), lambda i: (i, 0)),
                  pl.BlockSpec((block, H), lambda i: (i, 0))],
        out_specs=[pl.BlockSpec((block, H), lambda i: (i, 0)),
                   pl.BlockSpec((block, H), lambda i: (i, 0))],
        out_shape=[jax.ShapeDtypeStruct((m, H), jnp.float32),
                   jax.ShapeDtypeStruct((m, H), jnp.float32)],
    )(g, b)


def _round_body(a_ref, w_ref, mi_ref, o_ref):
    o_ref[...] = jnp.maximum(
        mi_ref[...] + jnp.dot(a_ref[...], w_ref[...],
                              preferred_element_type=jnp.float32), 0.0)


def _round_matmul(accum, w_h_t, msg_input, block):
    m = accum.shape[0]
    return pl.pallas_call(
        _round_body,
        grid=(m // block,),
        in_specs=[pl.BlockSpec((block, H), lambda i: (i, 0)),
                  pl.BlockSpec((H, H), lambda i: (0, 0)),
                  pl.BlockSpec((block, H), lambda i: (i, 0))],
        out_specs=pl.BlockSpec((block, H), lambda i: (i, 0)),
        out_shape=jax.ShapeDtypeStruct((m, H), jnp.float32),
    )(accum, w_h_t, msg_input)


def _final_body(atom_ref, m_ref, gid_ref, w1_ref, w2_ref, b_ref, o_ref,
                sums_ref, cnt_ref, *, nblocks, block, n_graphs):
    i = pl.program_id(0)

    @pl.when(i == 0)
    def _():
        sums_ref[...] = jnp.zeros_like(sums_ref)
        cnt_ref[...] = jnp.zeros_like(cnt_ref)

    h = jnp.maximum(
        jnp.dot(atom_ref[...], w1_ref[...], preferred_element_type=jnp.float32)
        + jnp.dot(m_ref[...], w2_ref[...], preferred_element_type=jnp.float32)
        + b_ref[...], 0.0)
    gids = gid_ref[...].reshape(block, 1)
    onehot = (gids == lax.broadcasted_iota(jnp.int32, (block, n_graphs), 1)
              ).astype(jnp.float32)
    sums_ref[...] += lax.dot_general(
        onehot, h, (((0,), (0,)), ((), ())),
        preferred_element_type=jnp.float32)
    cnt_ref[...] += jnp.sum(onehot, axis=0, keepdims=True)

    @pl.when(i == nblocks - 1)
    def _():
        counts = jnp.maximum(cnt_ref[...], 1.0).reshape(n_graphs, 1)
        o_ref[...] = sums_ref[...] / counts


def _final_readout(atom_features, m, graph_ids, w1_t, w2_t, bias, n_graphs,
                   block):
    n, fdim = atom_features.shape
    nblocks = n // block
    gid3 = graph_ids.reshape(nblocks, 1, block)
    body = functools.partial(_final_body, nblocks=nblocks, block=block,
                             n_graphs=n_graphs)
    return pl.pallas_call(
        body,
        grid=(nblocks,),
        in_specs=[pl.BlockSpec((block, fdim), lambda i: (i, 0)),
                  pl.BlockSpec((block, H), lambda i: (i, 0)),
                  pl.BlockSpec((1, 1, block), lambda i: (i, 0, 0)),
                  pl.BlockSpec((fdim, H), lambda i: (0, 0)),
                  pl.BlockSpec((H, H), lambda i: (0, 0)),
                  pl.BlockSpec((1, H), lambda i: (0, 0))],
        out_specs=pl.BlockSpec((n_graphs, H), lambda i: (0, 0)),
        out_shape=jax.ShapeDtypeStruct((n_graphs, H), jnp.float32),
        scratch_shapes=[pltpu.VMEM((n_graphs, H), jnp.float32),
                        pltpu.VMEM((1, n_graphs), jnp.float32)],
    )(atom_features, m, gid3, w1_t, w2_t, bias)


def _sc_gather(table, idx):
    e = idx.shape[0]
    per_w = e // (NC * NS)
    ch = 200
    assert per_w % ch == 0 and ch % 8 == 0
    mesh = plsc.VectorSubcoreMesh(core_axis_name="c", subcore_axis_name="s")

    @functools.partial(
        pl.kernel,
        out_type=jax.ShapeDtypeStruct((e, H), jnp.float32),
        mesh=mesh,
        compiler_params=pltpu.CompilerParams(needs_layout_passes=False),
        scratch_types=[pltpu.VMEM((ch,), jnp.int32),
                       pltpu.VMEM((ch, H), jnp.float32),
                       pltpu.SemaphoreType.DMA],
    )
    def k(table_hbm, idx_hbm, out_hbm, idx_v, rows_v, sem):
        wid = lax.axis_index("s") * NC + lax.axis_index("c")
        base = wid * per_w

        def body(j, carry):
            o = base + j * ch
            pltpu.sync_copy(idx_hbm.at[pl.ds(o, ch)], idx_v)
            pltpu.async_copy(table_hbm.at[idx_v], rows_v, sem).wait()
            pltpu.sync_copy(rows_v, out_hbm.at[pl.ds(o, ch)])
            return carry

        lax.fori_loop(0, per_w // ch, body, 0)

    return k(table, idx)


def _sc_segment_sum(data, gsrc, dst, out_rows, wr):
    j_tot = gsrc.shape[0]
    assert j_tot % NS == 0
    strip = j_tot // NS
    rp = -(-out_rows // (NS * 8)) * (NS * 8)
    nwin = -(-rp // wr)
    tail = rp - (nwin - 1) * wr
    br = 128
    ch2 = 2000
    cap = 2176
    nchunks = strip // ch2
    assert strip % ch2 == 0 and ch2 % LANES == 0 and cap % br == 0
    assert wr % (NS * 8) == 0 and tail % (NS * 8) == 0
    zb = 32
    assert (wr // NS) % zb == 0
    mesh = plsc.VectorSubcoreMesh(core_axis_name="c", subcore_axis_name="s")

    @functools.partial(
        pl.kernel,
        out_type=jax.ShapeDtypeStruct((rp, H), jnp.float32),
        mesh=mesh,
        compiler_params=pltpu.CompilerParams(needs_layout_passes=False),
        scratch_types=[
            pltpu.VMEM_SHARED((wr + LANES, H), jnp.float32),
            pltpu.VMEM((2 * ch2,), jnp.int32),
            pltpu.VMEM((2 * ch2,), jnp.int32),
            pltpu.VMEM((cap,), jnp.int32),
            pltpu.VMEM((cap // br, br), jnp.int32),
            pltpu.VMEM((2, br), jnp.int32),
            pltpu.VMEM((2, br), jnp.int32),
            pltpu.VMEM((2, br, H), jnp.float32),
            pltpu.VMEM((zb, H), jnp.float32),
            pltpu.SemaphoreType.DMA,
            pltpu.SemaphoreType.DMA,
            pltpu.SemaphoreType.DMA,
            pltpu.SemaphoreType.DMA,
            pltpu.SemaphoreType.DMA,
            pltpu.SemaphoreType.DMA,
        ],
    )
    def k(data_hbm, gsrc_hbm, dst_hbm, out_hbm, acc_sh, in_dst, in_gsrc,
          cg, cs, cgl, csl, rows_v, zbuf, isem0, isem1, gsem0, gsem1,
          ssem0, ssem1):
        cid = lax.axis_index("c")
        tid = lax.axis_index("s")
        isem = (isem0, isem1)
        gsem = (gsem0, gsem1)
        ssem = (ssem0, ssem1)

        def zrow(q, carry):
            r = q // (H // LANES)
            l = q % (H // LANES)
            zbuf[r, pl.ds(l * LANES, LANES)] = jnp.zeros((LANES,),
                                                         jnp.float32)
            return carry

        lax.fori_loop(0, zb * (H // LANES), zrow, 0)

        def fire_effects(b, fb):
            for s in (0, 1):
                @pl.when(fb % 2 == s)
                def _(s=s, b=b, fb=fb):
                    @pl.when(fb >= 1)
                    def _():
                        pltpu.make_async_copy(
                            data_hbm.at[cgl.at[1 - s]], rows_v.at[1 - s],
                            gsem[1 - s]).wait()
                        pltpu.async_copy(rows_v.at[1 - s],
                                         acc_sh.at[csl.at[1 - s]],
                                         ssem[1 - s], add=True)

                    @pl.when(fb >= 2)
                    def _():
                        pltpu.make_async_copy(rows_v.at[s],
                                              acc_sh.at[csl.at[s]],
                                              ssem[s]).wait()

                    for q in range(br // LANES):
                        cgl[s, pl.ds(q * LANES, LANES)] = (
                            cg[pl.ds(b * br + q * LANES, LANES)])
                        csl[s, pl.ds(q * LANES, LANES)] = (
                            cs[b, pl.ds(q * LANES, LANES)])
                    pltpu.async_copy(data_hbm.at[cgl.at[s]], rows_v.at[s],
                                     gsem[s])

        def process_window(w, wrows, share):
            base = w * wr
            for z in range(wr // NS // zb):
                pltpu.sync_copy(
                    zbuf, acc_sh.at[pl.ds(tid * (wr // NS) + z * zb, zb)])
            plsc.subcore_barrier()

            sbase = tid * strip
            pltpu.async_copy(dst_hbm.at[pl.ds(sbase, ch2)],
                             in_dst.at[pl.ds(0, ch2)], isem0)
            pltpu.async_copy(gsrc_hbm.at[pl.ds(sbase, ch2)],
                             in_gsrc.at[pl.ds(0, ch2)], isem0)

            def chunk(ci, carry):
                cnt, fb = carry
                for par in (0, 1):
                    @pl.when(ci % 2 == par)
                    def _(par=par, ci=ci):
                        @pl.when(ci + 1 < nchunks)
                        def _():
                            off2 = sbase + (ci + 1) * ch2
                            pltpu.async_copy(
                                dst_hbm.at[pl.ds(off2, ch2)],
                                in_dst.at[pl.ds((1 - par) * ch2, ch2)],
                                isem[1 - par])
                            pltpu.async_copy(
                                gsrc_hbm.at[pl.ds(off2, ch2)],
                                in_gsrc.at[pl.ds((1 - par) * ch2, ch2)],
                                isem[1 - par])
                        pltpu.make_async_copy(
                            dst_hbm.at[pl.ds(sbase, ch2)],
                            in_dst.at[pl.ds(par * ch2, ch2)],
                            isem[par]).wait()
                        pltpu.make_async_copy(
                            gsrc_hbm.at[pl.ds(sbase, ch2)],
                            in_gsrc.at[pl.ds(par * ch2, ch2)],
                            isem[par]).wait()
                cur = ci % 2

                def vreg(vi, cnt):
                    iota16 = lax.iota(jnp.int32, LANES)
                    d = in_dst[pl.ds(cur * ch2 + vi * LANES, LANES)]
                    g = in_gsrc[pl.ds(cur * ch2 + vi * LANES, LANES)]
                    msk = (d >= base) & (d < base + wrows)
                    pref = plsc.cumsum(msk.astype(jnp.int32))
                    pos = jnp.where(msk, cnt + pref - 1,
                                    cap - LANES + iota16)
                    plsc.store_scatter(cg, [pos], g)
                    plsc.store_scatter(cs, [pos // br, pos % br], d - base)
                    return cnt + pref[LANES - 1]

                cnt = lax.fori_loop(0, ch2 // LANES, vreg, cnt)

                nb = cnt // br

                def b_loop(b, fb):
                    fire_effects(b, fb)
                    return fb + 1

                fb = lax.fori_loop(0, nb, b_loop, fb)

                for q in range(br // LANES):
                    vg = cg[pl.ds(nb * br + q * LANES, LANES)]
                    cg[pl.ds(q * LANES, LANES)] = vg
                    vs = cs[nb, pl.ds(q * LANES, LANES)]
                    cs[0, pl.ds(q * LANES, LANES)] = vs
                return (cnt - nb * br, fb)

            cnt, fb = lax.fori_loop(0, nchunks, chunk,
                                    (jnp.int32(0), jnp.int32(0)))

            iota16 = lax.iota(jnp.int32, LANES)
            for p in range(br // LANES):
                pp = cnt + p * LANES + iota16
                plsc.store_scatter(cg, [pp], iota16)
                plsc.store_scatter(cs, [pp // br, pp % br], wr + iota16)

            @pl.when(cnt > 0)
            def _():
                fire_effects(0, fb)

            nf = fb + (cnt > 0).astype(jnp.int32)

            for s in (0, 1):
                @pl.when((nf >= 1) & ((nf - 1) % 2 == s))
                def _(s=s):
                    pltpu.make_async_copy(data_hbm.at[cgl.at[s]],
                                          rows_v.at[s], gsem[s]).wait()
                    pltpu.sync_copy(rows_v.at[s], acc_sh.at[csl.at[s]],
                                    add=True)

                @pl.when((nf >= 2) & (nf % 2 == s))
                def _(s=s):
                    pltpu.make_async_copy(rows_v.at[s],
                                          acc_sh.at[csl.at[s]],
                                          ssem[s]).wait()

            plsc.subcore_barrier()

            pltpu.sync_copy(acc_sh.at[pl.ds(tid * share, share)],
                            out_hbm.at[pl.ds(base + tid * share, share)])

        nfull = (nwin - 1 - cid + NC - 1) // NC

        def wloop(q, carry):
            process_window(q * NC + cid, wr, wr // NS)
            return carry

        lax.fori_loop(0, nfull, wloop, 0)

        @pl.when(cid == (nwin - 1) % NC)
        def _():
            process_window(nwin - 1, tail, tail // NS)

    res = k(data, gsrc, dst)
    return res if rp == out_rows else res[:out_rows]


def kernel(atom_features, bond_features, edge_index, line_edge_index,
           graph_ids, W_i, W_h, W_o_w, W_o_b):
    n, atom_fdim = atom_features.shape
    e = bond_features.shape[0]
    n_graphs = 64
    depth = 3

    src, dst = edge_index[0], edge_index[1]
    lsrc, ldst = line_edge_index[0], line_edge_index[1]

    w_i_a_t = W_i[:, :atom_fdim].T
    w_i_b_t = W_i[:, atom_fdim:].T
    w_h_t = W_h.T
    w_o_a_t = W_o_w[:, :atom_fdim].T
    w_o_m_t = W_o_w[:, atom_fdim:].T
    bias = W_o_b.reshape(1, H)

    a_proj = _matmul(atom_features, w_i_a_t, block=1000)
    b_proj = _matmul(bond_features, w_i_b_t, block=2000)
    g_rows = _sc_gather(a_proj, src)
    msg_input, msg = _add_relu(g_rows, b_proj, block=2000)

    for _ in range(depth - 1):
        accum = _sc_segment_sum(msg, lsrc, ldst, e, wr=9728)
        msg = _round_matmul(accum, w_h_t, msg_input, block=2000)

    edge_iota = jnp.arange(e, dtype=jnp.int32)
    m = _sc_segment_sum(msg, edge_iota, dst, n, wr=5120)

    return _final_readout(atom_features, m, graph_ids, w_o_a_t, w_o_m_t,
                          bias, n_graphs, block=1000)


# speedup vs baseline: 3.8533x; 1.0457x over previous
import functools

import jax
import jax.numpy as jnp
from jax import lax
from jax.experimental import pallas as pl
from jax.experimental.pallas import tpu as pltpu
from jax.experimental.pallas import tpu_sc as plsc

H = 128
NC = 2
NS = 16
LANES = 16


def _mm_body(x_ref, w_ref, o_ref):
    o_ref[...] = jnp.dot(x_ref[...], w_ref[...],
                         preferred_element_type=jnp.float32)


def _matmul(x, w_t, block):
    m, k = x.shape
    return pl.pallas_call(
        _mm_body,
        grid=(m // block,),
        in_specs=[pl.BlockSpec((block, k), lambda i: (i, 0)),
                  pl.BlockSpec((k, H), lambda i: (0, 0))],
        out_specs=pl.BlockSpec((block, H), lambda i: (i, 0)),
        out_shape=jax.ShapeDtypeStruct((m, H), jnp.float32),
    )(x, w_t)


def _add_relu_body(g_ref, bond_ref, w_ref, mi_ref, msg_ref):
    s = g_ref[...] + jnp.dot(bond_ref[...], w_ref[...],
                             preferred_element_type=jnp.float32)
    mi_ref[...] = s
    msg_ref[...] = jnp.maximum(s, 0.0)


def _add_relu(g, bond, w_b_t, block):
    m, kb = bond.shape
    return pl.pallas_call(
        _add_relu_body,
        grid=(m // block,),
        in_specs=[pl.BlockSpec((block, H), lambda i: (i, 0)),
                  pl.BlockSpec((block, kb), lambda i: (i, 0)),
                  pl.BlockSpec((kb, H), lambda i: (0, 0))],
        out_specs=[pl.BlockSpec((block, H), lambda i: (i, 0)),
                   pl.BlockSpec((block, H), lambda i: (i, 0))],
        out_shape=[jax.ShapeDtypeStruct((m, H), jnp.float32),
                   jax.ShapeDtypeStruct((m, H), jnp.float32)],
    )(g, bond, w_b_t)


def _round_body(a_ref, w_ref, mi_ref, o_ref):
    o_ref[...] = jnp.maximum(
        mi_ref[...] + jnp.dot(a_ref[...], w_ref[...],
                              preferred_element_type=jnp.float32), 0.0)


def _round_matmul(accum, w_h_t, msg_input, block):
    m = accum.shape[0]
    return pl.pallas_call(
        _round_body,
        grid=(m // block,),
        in_specs=[pl.BlockSpec((block, H), lambda i: (i, 0)),
                  pl.BlockSpec((H, H), lambda i: (0, 0)),
                  pl.BlockSpec((block, H), lambda i: (i, 0))],
        out_specs=pl.BlockSpec((block, H), lambda i: (i, 0)),
        out_shape=jax.ShapeDtypeStruct((m, H), jnp.float32),
    )(accum, w_h_t, msg_input)


def _final_body(atom_ref, m_ref, gid_ref, w1_ref, w2_ref, b_ref, o_ref,
                sums_ref, cnt_ref, *, nblocks, block, n_graphs):
    i = pl.program_id(0)

    @pl.when(i == 0)
    def _():
        sums_ref[...] = jnp.zeros_like(sums_ref)
        cnt_ref[...] = jnp.zeros_like(cnt_ref)

    h = jnp.maximum(
        jnp.dot(atom_ref[...], w1_ref[...], preferred_element_type=jnp.float32)
        + jnp.dot(m_ref[...], w2_ref[...], preferred_element_type=jnp.float32)
        + b_ref[...], 0.0)
    gids = gid_ref[...].reshape(block, 1)
    onehot = (gids == lax.broadcasted_iota(jnp.int32, (block, n_graphs), 1)
              ).astype(jnp.float32)
    sums_ref[...] += lax.dot_general(
        onehot, h, (((0,), (0,)), ((), ())),
        preferred_element_type=jnp.float32)
    cnt_ref[...] += jnp.sum(onehot, axis=0, keepdims=True)

    @pl.when(i == nblocks - 1)
    def _():
        counts = jnp.maximum(cnt_ref[...], 1.0).reshape(n_graphs, 1)
        o_ref[...] = sums_ref[...] / counts


def _final_readout(atom_features, m, graph_ids, w1_t, w2_t, bias, n_graphs,
                   block):
    n, fdim = atom_features.shape
    nblocks = n // block
    gid3 = graph_ids.reshape(nblocks, 1, block)
    body = functools.partial(_final_body, nblocks=nblocks, block=block,
                             n_graphs=n_graphs)
    return pl.pallas_call(
        body,
        grid=(nblocks,),
        in_specs=[pl.BlockSpec((block, fdim), lambda i: (i, 0)),
                  pl.BlockSpec((block, H), lambda i: (i, 0)),
                  pl.BlockSpec((1, 1, block), lambda i: (i, 0, 0)),
                  pl.BlockSpec((fdim, H), lambda i: (0, 0)),
                  pl.BlockSpec((H, H), lambda i: (0, 0)),
                  pl.BlockSpec((1, H), lambda i: (0, 0))],
        out_specs=pl.BlockSpec((n_graphs, H), lambda i: (0, 0)),
        out_shape=jax.ShapeDtypeStruct((n_graphs, H), jnp.float32),
        scratch_shapes=[pltpu.VMEM((n_graphs, H), jnp.float32),
                        pltpu.VMEM((1, n_graphs), jnp.float32)],
    )(atom_features, m, gid3, w1_t, w2_t, bias)


def _sc_gather(table, idx):
    e = idx.shape[0]
    per_w = e // (NC * NS)
    ch = 200
    nit = per_w // ch
    assert per_w % ch == 0 and ch % 8 == 0 and nit >= 2
    mesh = plsc.VectorSubcoreMesh(core_axis_name="c", subcore_axis_name="s")

    @functools.partial(
        pl.kernel,
        out_type=jax.ShapeDtypeStruct((e, H), jnp.float32),
        mesh=mesh,
        compiler_params=pltpu.CompilerParams(needs_layout_passes=False),
        scratch_types=[pltpu.VMEM((2 * ch,), jnp.int32),
                       pltpu.VMEM((ch, H), jnp.float32),
                       pltpu.VMEM((ch, H), jnp.float32),
                       pltpu.SemaphoreType.DMA,
                       pltpu.SemaphoreType.DMA,
                       pltpu.SemaphoreType.DMA,
                       pltpu.SemaphoreType.DMA,
                       pltpu.SemaphoreType.DMA,
                       pltpu.SemaphoreType.DMA],
    )
    def k(table_hbm, idx_hbm, out_hbm, idx_v, rows0, rows1,
          x0, x1, g0, g1, w0, w1):
        xsem, gsem, wsem = (x0, x1), (g0, g1), (w0, w1)
        rows = (rows0, rows1)
        wid = lax.axis_index("s") * NC + lax.axis_index("c")
        base = wid * per_w
        pltpu.async_copy(idx_hbm.at[pl.ds(base, ch)],
                         idx_v.at[pl.ds(0, ch)], x0)

        def body(j, carry):
            for s in (0, 1):
                @pl.when(j % 2 == s)
                def _(s=s, j=j):
                    @pl.when(j + 1 < nit)
                    def _():
                        pltpu.async_copy(
                            idx_hbm.at[pl.ds(base + (j + 1) * ch, ch)],
                            idx_v.at[pl.ds((1 - s) * ch, ch)], xsem[1 - s])
                    pltpu.make_async_copy(
                        idx_hbm.at[pl.ds(base, ch)],
                        idx_v.at[pl.ds(s * ch, ch)], xsem[s]).wait()

                    @pl.when(j >= 2)
                    def _():
                        pltpu.make_async_copy(
                            rows[s], out_hbm.at[pl.ds(base, ch)],
                            wsem[s]).wait()

                    pltpu.async_copy(
                        table_hbm.at[idx_v.at[pl.ds(s * ch, ch)]],
                        rows[s], gsem[s])

                    @pl.when(j >= 1)
                    def _():
                        pltpu.make_async_copy(
                            table_hbm.at[idx_v.at[pl.ds((1 - s) * ch, ch)]],
                            rows[1 - s], gsem[1 - s]).wait()
                        pltpu.async_copy(
                            rows[1 - s],
                            out_hbm.at[pl.ds(base + (j - 1) * ch, ch)],
                            wsem[1 - s])
            return carry

        lax.fori_loop(0, nit, body, 0)

        sl = (nit - 1) % 2
        pltpu.make_async_copy(table_hbm.at[idx_v.at[pl.ds(sl * ch, ch)]],
                              rows[sl], gsem[sl]).wait()
        pltpu.async_copy(rows[sl],
                         out_hbm.at[pl.ds(base + (nit - 1) * ch, ch)],
                         wsem[sl]).wait()
        s2 = (nit - 2) % 2
        pltpu.make_async_copy(rows[s2], out_hbm.at[pl.ds(base, ch)],
                              wsem[s2]).wait()

    return k(table, idx)


def _sc_segment_sum(data, gsrc, dst, out_rows, wr):
    j_tot = gsrc.shape[0]
    assert j_tot % NS == 0
    strip = j_tot // NS
    rp = -(-out_rows // (NS * 8)) * (NS * 8)
    nwin = -(-rp // wr)
    tail = rp - (nwin - 1) * wr
    br = 128
    ch2 = 2000
    cap = 2176
    nchunks = strip // ch2
    assert strip % ch2 == 0 and ch2 % LANES == 0 and cap % br == 0
    assert wr % (NS * 8) == 0 and tail % (NS * 8) == 0
    zb = 32
    assert (wr // NS) % zb == 0
    mesh = plsc.VectorSubcoreMesh(core_axis_name="c", subcore_axis_name="s")

    @functools.partial(
        pl.kernel,
        out_type=jax.ShapeDtypeStruct((rp, H), jnp.float32),
        mesh=mesh,
        compiler_params=pltpu.CompilerParams(needs_layout_passes=False),
        scratch_types=[
            pltpu.VMEM_SHARED((wr + LANES, H), jnp.float32),
            pltpu.VMEM((2 * ch2,), jnp.int32),
            pltpu.VMEM((2 * ch2,), jnp.int32),
            pltpu.VMEM((cap,), jnp.int32),
            pltpu.VMEM((cap // br, br), jnp.int32),
            pltpu.VMEM((2, br), jnp.int32),
            pltpu.VMEM((2, br), jnp.int32),
            pltpu.VMEM((2, br, H), jnp.float32),
            pltpu.VMEM((zb, H), jnp.float32),
            pltpu.SemaphoreType.DMA,
            pltpu.SemaphoreType.DMA,
            pltpu.SemaphoreType.DMA,
            pltpu.SemaphoreType.DMA,
            pltpu.SemaphoreType.DMA,
            pltpu.SemaphoreType.DMA,
        ],
    )
    def k(data_hbm, gsrc_hbm, dst_hbm, out_hbm, acc_sh, in_dst, in_gsrc,
          cg, cs, cgl, csl, rows_v, zbuf, isem0, isem1, gsem0, gsem1,
          ssem0, ssem1):
        cid = lax.axis_index("c")
        tid = lax.axis_index("s")
        isem = (isem0, isem1)
        gsem = (gsem0, gsem1)
        ssem = (ssem0, ssem1)

        def zrow(q, carry):
            r = q // (H // LANES)
            l = q % (H // LANES)
            zbuf[r, pl.ds(l * LANES, LANES)] = jnp.zeros((LANES,),
                                                         jnp.float32)
            return carry

        lax.fori_loop(0, zb * (H // LANES), zrow, 0)

        def fire_effects(b, fb):
            for s in (0, 1):
                @pl.when(fb % 2 == s)
                def _(s=s, b=b, fb=fb):
                    @pl.when(fb >= 1)
                    def _():
                        pltpu.make_async_copy(
                            data_hbm.at[cgl.at[1 - s]], rows_v.at[1 - s],
                            gsem[1 - s]).wait()
                        pltpu.async_copy(rows_v.at[1 - s],
                                         acc_sh.at[csl.at[1 - s]],
                                         ssem[1 - s], add=True)

                    @pl.when(fb >= 2)
                    def _():
                        pltpu.make_async_copy(rows_v.at[s],
                                              acc_sh.at[csl.at[s]],
                                              ssem[s]).wait()

                    for q in range(br // LANES):
                        cgl[s, pl.ds(q * LANES, LANES)] = (
                            cg[pl.ds(b * br + q * LANES, LANES)])
                        csl[s, pl.ds(q * LANES, LANES)] = (
                            cs[b, pl.ds(q * LANES, LANES)])
                    pltpu.async_copy(data_hbm.at[cgl.at[s]], rows_v.at[s],
                                     gsem[s])

        def process_window(w, wrows, share):
            base = w * wr
            for z in range(wr // NS // zb):
                pltpu.sync_copy(
                    zbuf, acc_sh.at[pl.ds(tid * (wr // NS) + z * zb, zb)])
            plsc.subcore_barrier()

            sbase = tid * strip
            pltpu.async_copy(dst_hbm.at[pl.ds(sbase, ch2)],
                             in_dst.at[pl.ds(0, ch2)], isem0)
            pltpu.async_copy(gsrc_hbm.at[pl.ds(sbase, ch2)],
                             in_gsrc.at[pl.ds(0, ch2)], isem0)

            def chunk(ci, carry):
                cnt, fb = carry
                for par in (0, 1):
                    @pl.when(ci % 2 == par)
                    def _(par=par, ci=ci):
                        @pl.when(ci + 1 < nchunks)
                        def _():
                            off2 = sbase + (ci + 1) * ch2
                            pltpu.async_copy(
                                dst_hbm.at[pl.ds(off2, ch2)],
                                in_dst.at[pl.ds((1 - par) * ch2, ch2)],
                                isem[1 - par])
                            pltpu.async_copy(
                                gsrc_hbm.at[pl.ds(off2, ch2)],
                                in_gsrc.at[pl.ds((1 - par) * ch2, ch2)],
                                isem[1 - par])
                        pltpu.make_async_copy(
                            dst_hbm.at[pl.ds(sbase, ch2)],
                            in_dst.at[pl.ds(par * ch2, ch2)],
                            isem[par]).wait()
                        pltpu.make_async_copy(
                            gsrc_hbm.at[pl.ds(sbase, ch2)],
                            in_gsrc.at[pl.ds(par * ch2, ch2)],
                            isem[par]).wait()
                cur = ci % 2

                def vreg(vi, cnt):
                    iota16 = lax.iota(jnp.int32, LANES)
                    d = in_dst[pl.ds(cur * ch2 + vi * LANES, LANES)]
                    g = in_gsrc[pl.ds(cur * ch2 + vi * LANES, LANES)]
                    msk = (d >= base) & (d < base + wrows)
                    pref = plsc.cumsum(msk.astype(jnp.int32))
                    pos = jnp.where(msk, cnt + pref - 1,
                                    cap - LANES + iota16)
                    plsc.store_scatter(cg, [pos], g)
                    plsc.store_scatter(cs, [pos // br, pos % br], d - base)
                    return cnt + pref[LANES - 1]

                cnt = lax.fori_loop(0, ch2 // LANES, vreg, cnt)

                nb = cnt // br

                def b_loop(b, fb):
                    fire_effects(b, fb)
                    return fb + 1

                fb = lax.fori_loop(0, nb, b_loop, fb)

                for q in range(br // LANES):
                    vg = cg[pl.ds(nb * br + q * LANES, LANES)]
                    cg[pl.ds(q * LANES, LANES)] = vg
                    vs = cs[nb, pl.ds(q * LANES, LANES)]
                    cs[0, pl.ds(q * LANES, LANES)] = vs
                return (cnt - nb * br, fb)

            cnt, fb = lax.fori_loop(0, nchunks, chunk,
                                    (jnp.int32(0), jnp.int32(0)))

            iota16 = lax.iota(jnp.int32, LANES)
            for p in range(br // LANES):
                pp = cnt + p * LANES + iota16
                plsc.store_scatter(cg, [pp], iota16)
                plsc.store_scatter(cs, [pp // br, pp % br], wr + iota16)

            @pl.when(cnt > 0)
            def _():
                fire_effects(0, fb)

            nf = fb + (cnt > 0).astype(jnp.int32)

            for s in (0, 1):
                @pl.when((nf >= 1) & ((nf - 1) % 2 == s))
                def _(s=s):
                    pltpu.make_async_copy(data_hbm.at[cgl.at[s]],
                                          rows_v.at[s], gsem[s]).wait()
                    pltpu.sync_copy(rows_v.at[s], acc_sh.at[csl.at[s]],
                                    add=True)

                @pl.when((nf >= 2) & (nf % 2 == s))
                def _(s=s):
                    pltpu.make_async_copy(rows_v.at[s],
                                          acc_sh.at[csl.at[s]],
                                          ssem[s]).wait()

            plsc.subcore_barrier()

            pltpu.sync_copy(acc_sh.at[pl.ds(tid * share, share)],
                            out_hbm.at[pl.ds(base + tid * share, share)])

        nfull = (nwin - 1 - cid + NC - 1) // NC

        def wloop(q, carry):
            process_window(q * NC + cid, wr, wr // NS)
            return carry

        lax.fori_loop(0, nfull, wloop, 0)

        @pl.when(cid == (nwin - 1) % NC)
        def _():
            process_window(nwin - 1, tail, tail // NS)

    res = k(data, gsrc, dst)
    return res if rp == out_rows else res[:out_rows]


def kernel(atom_features, bond_features, edge_index, line_edge_index,
           graph_ids, W_i, W_h, W_o_w, W_o_b):
    n, atom_fdim = atom_features.shape
    e = bond_features.shape[0]
    n_graphs = 64
    depth = 3

    src, dst = edge_index[0], edge_index[1]
    lsrc, ldst = line_edge_index[0], line_edge_index[1]

    w_i_a_t = W_i[:, :atom_fdim].T
    w_i_b_t = W_i[:, atom_fdim:].T
    w_h_t = W_h.T
    w_o_a_t = W_o_w[:, :atom_fdim].T
    w_o_m_t = W_o_w[:, atom_fdim:].T
    bias = W_o_b.reshape(1, H)

    a_proj = _matmul(atom_features, w_i_a_t, block=1000)
    g_rows = _sc_gather(a_proj, src)
    msg_input, msg = _add_relu(g_rows, bond_features, w_i_b_t, block=2000)

    for _ in range(depth - 1):
        accum = _sc_segment_sum(msg, lsrc, ldst, e, wr=9728)
        msg = _round_matmul(accum, w_h_t, msg_input, block=2000)

    edge_iota = jnp.arange(e, dtype=jnp.int32)
    m = _sc_segment_sum(msg, edge_iota, dst, n, wr=5120)

    return _final_readout(atom_features, m, graph_ids, w_o_a_t, w_o_m_t,
                          bias, n_graphs, block=1000)


# speedup vs baseline: 3.9004x; 1.0122x over previous
import functools

import jax
import jax.numpy as jnp
from jax import lax
from jax.experimental import pallas as pl
from jax.experimental.pallas import tpu as pltpu
from jax.experimental.pallas import tpu_sc as plsc

H = 128
NC = 2
NS = 16
LANES = 16


def _mm_body(x_ref, w_ref, o_ref):
    o_ref[...] = jnp.dot(x_ref[...], w_ref[...],
                         preferred_element_type=jnp.float32)


def _matmul(x, w_t, block):
    m, k = x.shape
    return pl.pallas_call(
        _mm_body,
        grid=(m // block,),
        in_specs=[pl.BlockSpec((block, k), lambda i: (i, 0)),
                  pl.BlockSpec((k, H), lambda i: (0, 0))],
        out_specs=pl.BlockSpec((block, H), lambda i: (i, 0)),
        out_shape=jax.ShapeDtypeStruct((m, H), jnp.float32),
    )(x, w_t)


def _add_relu_body(g_ref, bond_ref, w_ref, mi_ref, msg_ref):
    s = g_ref[...] + jnp.dot(bond_ref[...], w_ref[...],
                             preferred_element_type=jnp.float32)
    mi_ref[...] = s
    msg_ref[...] = jnp.maximum(s, 0.0)


def _add_relu(g, bond, w_b_t, block):
    m, kb = bond.shape
    return pl.pallas_call(
        _add_relu_body,
        grid=(m // block,),
        in_specs=[pl.BlockSpec((block, H), lambda i: (i, 0)),
                  pl.BlockSpec((block, kb), lambda i: (i, 0)),
                  pl.BlockSpec((kb, H), lambda i: (0, 0))],
        out_specs=[pl.BlockSpec((block, H), lambda i: (i, 0)),
                   pl.BlockSpec((block, H), lambda i: (i, 0))],
        out_shape=[jax.ShapeDtypeStruct((m, H), jnp.float32),
                   jax.ShapeDtypeStruct((m, H), jnp.float32)],
    )(g, bond, w_b_t)


def _round_body(a_ref, w_ref, mi_ref, o_ref):
    o_ref[...] = jnp.maximum(
        mi_ref[...] + jnp.dot(a_ref[...], w_ref[...],
                              preferred_element_type=jnp.float32), 0.0)


def _round_matmul(accum, w_h_t, msg_input, block):
    m = accum.shape[0]
    return pl.pallas_call(
        _round_body,
        grid=(m // block,),
        in_specs=[pl.BlockSpec((block, H), lambda i: (i, 0)),
                  pl.BlockSpec((H, H), lambda i: (0, 0)),
                  pl.BlockSpec((block, H), lambda i: (i, 0))],
        out_specs=pl.BlockSpec((block, H), lambda i: (i, 0)),
        out_shape=jax.ShapeDtypeStruct((m, H), jnp.float32),
    )(accum, w_h_t, msg_input)


def _final_body(atom_ref, m_ref, gid_ref, w1_ref, w2_ref, b_ref, o_ref,
                sums_ref, cnt_ref, *, nblocks, block, n_graphs):
    i = pl.program_id(0)

    @pl.when(i == 0)
    def _():
        sums_ref[...] = jnp.zeros_like(sums_ref)
        cnt_ref[...] = jnp.zeros_like(cnt_ref)

    h = jnp.maximum(
        jnp.dot(atom_ref[...], w1_ref[...], preferred_element_type=jnp.float32)
        + jnp.dot(m_ref[...], w2_ref[...], preferred_element_type=jnp.float32)
        + b_ref[...], 0.0)
    gids = gid_ref[...].reshape(block, 1)
    onehot = (gids == lax.broadcasted_iota(jnp.int32, (block, n_graphs), 1)
              ).astype(jnp.float32)
    sums_ref[...] += lax.dot_general(
        onehot, h, (((0,), (0,)), ((), ())),
        preferred_element_type=jnp.float32)
    cnt_ref[...] += jnp.sum(onehot, axis=0, keepdims=True)

    @pl.when(i == nblocks - 1)
    def _():
        counts = jnp.maximum(cnt_ref[...], 1.0).reshape(n_graphs, 1)
        o_ref[...] = sums_ref[...] / counts


def _final_readout(atom_features, m, graph_ids, w1_t, w2_t, bias, n_graphs,
                   block):
    n, fdim = atom_features.shape
    nblocks = n // block
    gid3 = graph_ids.reshape(nblocks, 1, block)
    body = functools.partial(_final_body, nblocks=nblocks, block=block,
                             n_graphs=n_graphs)
    return pl.pallas_call(
        body,
        grid=(nblocks,),
        in_specs=[pl.BlockSpec((block, fdim), lambda i: (i, 0)),
                  pl.BlockSpec((block, H), lambda i: (i, 0)),
                  pl.BlockSpec((1, 1, block), lambda i: (i, 0, 0)),
                  pl.BlockSpec((fdim, H), lambda i: (0, 0)),
                  pl.BlockSpec((H, H), lambda i: (0, 0)),
                  pl.BlockSpec((1, H), lambda i: (0, 0))],
        out_specs=pl.BlockSpec((n_graphs, H), lambda i: (0, 0)),
        out_shape=jax.ShapeDtypeStruct((n_graphs, H), jnp.float32),
        scratch_shapes=[pltpu.VMEM((n_graphs, H), jnp.float32),
                        pltpu.VMEM((1, n_graphs), jnp.float32)],
    )(atom_features, m, gid3, w1_t, w2_t, bias)


def _sc_gather(table, idx):
    e = idx.shape[0]
    per_w = e // (NC * NS)
    ch = 200
    nit = per_w // ch
    assert per_w % ch == 0 and ch % 8 == 0 and nit >= 2
    mesh = plsc.VectorSubcoreMesh(core_axis_name="c", subcore_axis_name="s")

    @functools.partial(
        pl.kernel,
        out_type=jax.ShapeDtypeStruct((e, H), jnp.float32),
        mesh=mesh,
        compiler_params=pltpu.CompilerParams(needs_layout_passes=False),
        scratch_types=[pltpu.VMEM((2 * ch,), jnp.int32),
                       pltpu.VMEM((ch, H), jnp.float32),
                       pltpu.VMEM((ch, H), jnp.float32),
                       pltpu.SemaphoreType.DMA,
                       pltpu.SemaphoreType.DMA,
                       pltpu.SemaphoreType.DMA,
                       pltpu.SemaphoreType.DMA,
                       pltpu.SemaphoreType.DMA,
                       pltpu.SemaphoreType.DMA],
    )
    def k(table_hbm, idx_hbm, out_hbm, idx_v, rows0, rows1,
          x0, x1, g0, g1, w0, w1):
        xsem, gsem, wsem = (x0, x1), (g0, g1), (w0, w1)
        rows = (rows0, rows1)
        wid = lax.axis_index("s") * NC + lax.axis_index("c")
        base = wid * per_w
        pltpu.async_copy(idx_hbm.at[pl.ds(base, ch)],
                         idx_v.at[pl.ds(0, ch)], x0)

        def body(j, carry):
            for s in (0, 1):
                @pl.when(j % 2 == s)
                def _(s=s, j=j):
                    @pl.when(j >= 1)
                    def _():
                        pltpu.make_async_copy(
                            table_hbm.at[idx_v.at[pl.ds((1 - s) * ch, ch)]],
                            rows[1 - s], gsem[1 - s]).wait()
                        pltpu.async_copy(
                            rows[1 - s],
                            out_hbm.at[pl.ds(base + (j - 1) * ch, ch)],
                            wsem[1 - s])

                    @pl.when(j + 1 < nit)
                    def _():
                        pltpu.async_copy(
                            idx_hbm.at[pl.ds(base + (j + 1) * ch, ch)],
                            idx_v.at[pl.ds((1 - s) * ch, ch)], xsem[1 - s])
                    pltpu.make_async_copy(
                        idx_hbm.at[pl.ds(base, ch)],
                        idx_v.at[pl.ds(s * ch, ch)], xsem[s]).wait()

                    @pl.when(j >= 2)
                    def _():
                        pltpu.make_async_copy(
                            rows[s], out_hbm.at[pl.ds(base, ch)],
                            wsem[s]).wait()

                    pltpu.async_copy(
                        table_hbm.at[idx_v.at[pl.ds(s * ch, ch)]],
                        rows[s], gsem[s])
            return carry

        lax.fori_loop(0, nit, body, 0)

        sl = (nit - 1) % 2
        pltpu.make_async_copy(table_hbm.at[idx_v.at[pl.ds(sl * ch, ch)]],
                              rows[sl], gsem[sl]).wait()
        pltpu.async_copy(rows[sl],
                         out_hbm.at[pl.ds(base + (nit - 1) * ch, ch)],
                         wsem[sl]).wait()
        s2 = (nit - 2) % 2
        pltpu.make_async_copy(rows[s2], out_hbm.at[pl.ds(base, ch)],
                              wsem[s2]).wait()

    return k(table, idx)


def _sc_segment_sum(data, gsrc, dst, out_rows, wr, ch2=2000, unroll=1):
    j_tot = gsrc.shape[0]
    assert j_tot % NS == 0
    strip = j_tot // NS
    rp = -(-out_rows // (NS * 8)) * (NS * 8)
    nwin = -(-rp // wr)
    tail = rp - (nwin - 1) * wr
    br = 128
    cap = -(-(ch2 + 2 * br + LANES) // br) * br
    nchunks = strip // ch2
    assert strip % ch2 == 0 and ch2 % (LANES * unroll) == 0 and cap % br == 0
    assert wr % (NS * 8) == 0 and tail % (NS * 8) == 0
    zb = 32
    assert (wr // NS) % zb == 0
    mesh = plsc.VectorSubcoreMesh(core_axis_name="c", subcore_axis_name="s")

    @functools.partial(
        pl.kernel,
        out_type=jax.ShapeDtypeStruct((rp, H), jnp.float32),
        mesh=mesh,
        compiler_params=pltpu.CompilerParams(needs_layout_passes=False),
        scratch_types=[
            pltpu.VMEM_SHARED((wr + LANES, H), jnp.float32),
            pltpu.VMEM((2 * ch2,), jnp.int32),
            pltpu.VMEM((2 * ch2,), jnp.int32),
            pltpu.VMEM((cap,), jnp.int32),
            pltpu.VMEM((cap // br, br), jnp.int32),
            pltpu.VMEM((2, br), jnp.int32),
            pltpu.VMEM((2, br), jnp.int32),
            pltpu.VMEM((2, br, H), jnp.float32),
            pltpu.VMEM((zb, H), jnp.float32),
            pltpu.SemaphoreType.DMA,
            pltpu.SemaphoreType.DMA,
            pltpu.SemaphoreType.DMA,
            pltpu.SemaphoreType.DMA,
            pltpu.SemaphoreType.DMA,
            pltpu.SemaphoreType.DMA,
        ],
    )
    def k(data_hbm, gsrc_hbm, dst_hbm, out_hbm, acc_sh, in_dst, in_gsrc,
          cg, cs, cgl, csl, rows_v, zbuf, isem0, isem1, gsem0, gsem1,
          ssem0, ssem1):
        cid = lax.axis_index("c")
        tid = lax.axis_index("s")
        isem = (isem0, isem1)
        gsem = (gsem0, gsem1)
        ssem = (ssem0, ssem1)

        def zrow(q, carry):
            r = q // (H // LANES)
            l = q % (H // LANES)
            zbuf[r, pl.ds(l * LANES, LANES)] = jnp.zeros((LANES,),
                                                         jnp.float32)
            return carry

        lax.fori_loop(0, zb * (H // LANES), zrow, 0)

        def fire_effects(b, fb):
            for s in (0, 1):
                @pl.when(fb % 2 == s)
                def _(s=s, b=b, fb=fb):
                    @pl.when(fb >= 1)
                    def _():
                        pltpu.make_async_copy(
                            data_hbm.at[cgl.at[1 - s]], rows_v.at[1 - s],
                            gsem[1 - s]).wait()
                        pltpu.async_copy(rows_v.at[1 - s],
                                         acc_sh.at[csl.at[1 - s]],
                                         ssem[1 - s], add=True)

                    @pl.when(fb >= 2)
                    def _():
                        pltpu.make_async_copy(rows_v.at[s],
                                              acc_sh.at[csl.at[s]],
                                              ssem[s]).wait()

                    for q in range(br // LANES):
                        cgl[s, pl.ds(q * LANES, LANES)] = (
                            cg[pl.ds(b * br + q * LANES, LANES)])
                        csl[s, pl.ds(q * LANES, LANES)] = (
                            cs[b, pl.ds(q * LANES, LANES)])
                    pltpu.async_copy(data_hbm.at[cgl.at[s]], rows_v.at[s],
                                     gsem[s])

        def process_window(w, wrows, share):
            base = w * wr
            for z in range(wr // NS // zb):
                pltpu.sync_copy(
                    zbuf, acc_sh.at[pl.ds(tid * (wr // NS) + z * zb, zb)])
            plsc.subcore_barrier()

            sbase = tid * strip
            pltpu.async_copy(dst_hbm.at[pl.ds(sbase, ch2)],
                             in_dst.at[pl.ds(0, ch2)], isem0)
            pltpu.async_copy(gsrc_hbm.at[pl.ds(sbase, ch2)],
                             in_gsrc.at[pl.ds(0, ch2)], isem0)

            def chunk(ci, carry):
                cnt, fb = carry
                for par in (0, 1):
                    @pl.when(ci % 2 == par)
                    def _(par=par, ci=ci):
                        @pl.when(ci + 1 < nchunks)
                        def _():
                            off2 = sbase + (ci + 1) * ch2
                            pltpu.async_copy(
                                dst_hbm.at[pl.ds(off2, ch2)],
                                in_dst.at[pl.ds((1 - par) * ch2, ch2)],
                                isem[1 - par])
                            pltpu.async_copy(
                                gsrc_hbm.at[pl.ds(off2, ch2)],
                                in_gsrc.at[pl.ds((1 - par) * ch2, ch2)],
                                isem[1 - par])
                        pltpu.make_async_copy(
                            dst_hbm.at[pl.ds(sbase, ch2)],
                            in_dst.at[pl.ds(par * ch2, ch2)],
                            isem[par]).wait()
                        pltpu.make_async_copy(
                            gsrc_hbm.at[pl.ds(sbase, ch2)],
                            in_gsrc.at[pl.ds(par * ch2, ch2)],
                            isem[par]).wait()
                cur = ci % 2

                def vreg(vi, cnt):
                    iota16 = lax.iota(jnp.int32, LANES)
                    off0 = cur * ch2 + vi * (LANES * unroll)
                    for u in range(unroll):
                        d = in_dst[pl.ds(off0 + u * LANES, LANES)]
                        g = in_gsrc[pl.ds(off0 + u * LANES, LANES)]
                        msk = (d >= base) & (d < base + wrows)
                        pref = plsc.cumsum(msk.astype(jnp.int32))
                        pos = jnp.where(msk, cnt + pref - 1,
                                        cap - LANES + iota16)
                        plsc.store_scatter(cg, [pos], g)
                        plsc.store_scatter(cs, [pos // br, pos % br],
                                           d - base)
                        cnt = cnt + pref[LANES - 1]
                    return cnt

                cnt = lax.fori_loop(0, ch2 // (LANES * unroll), vreg, cnt)

                nb = cnt // br

                def b_loop(b, fb):
                    fire_effects(b, fb)
                    return fb + 1

                fb = lax.fori_loop(0, nb, b_loop, fb)

                for q in range(br // LANES):
                    vg = cg[pl.ds(nb * br + q * LANES, LANES)]
                    cg[pl.ds(q * LANES, LANES)] = vg
                    vs = cs[nb, pl.ds(q * LANES, LANES)]
                    cs[0, pl.ds(q * LANES, LANES)] = vs
                return (cnt - nb * br, fb)

            cnt, fb = lax.fori_loop(0, nchunks, chunk,
                                    (jnp.int32(0), jnp.int32(0)))

            iota16 = lax.iota(jnp.int32, LANES)
            for p in range(br // LANES):
                pp = cnt + p * LANES + iota16
                plsc.store_scatter(cg, [pp], iota16)
                plsc.store_scatter(cs, [pp // br, pp % br], wr + iota16)

            @pl.when(cnt > 0)
            def _():
                fire_effects(0, fb)

            nf = fb + (cnt > 0).astype(jnp.int32)

            for s in (0, 1):
                @pl.when((nf >= 1) & ((nf - 1) % 2 == s))
                def _(s=s):
                    pltpu.make_async_copy(data_hbm.at[cgl.at[s]],
                                          rows_v.at[s], gsem[s]).wait()
                    pltpu.sync_copy(rows_v.at[s], acc_sh.at[csl.at[s]],
                                    add=True)

                @pl.when((nf >= 2) & (nf % 2 == s))
                def _(s=s):
                    pltpu.make_async_copy(rows_v.at[s],
                                          acc_sh.at[csl.at[s]],
                                          ssem[s]).wait()

            plsc.subcore_barrier()

            pltpu.sync_copy(acc_sh.at[pl.ds(tid * share, share)],
                            out_hbm.at[pl.ds(base + tid * share, share)])

        nfull = (nwin - 1 - cid + NC - 1) // NC

        def wloop(q, carry):
            process_window(q * NC + cid, wr, wr // NS)
            return carry

        lax.fori_loop(0, nfull, wloop, 0)

        @pl.when(cid == (nwin - 1) % NC)
        def _():
            process_window(nwin - 1, tail, tail // NS)

    res = k(data, gsrc, dst)
    return res if rp == out_rows else res[:out_rows]


def kernel(atom_features, bond_features, edge_index, line_edge_index,
           graph_ids, W_i, W_h, W_o_w, W_o_b):
    n, atom_fdim = atom_features.shape
    e = bond_features.shape[0]
    n_graphs = 64
    depth = 3

    src, dst = edge_index[0], edge_index[1]
    lsrc, ldst = line_edge_index[0], line_edge_index[1]

    w_i_a_t = W_i[:, :atom_fdim].T
    w_i_b_t = W_i[:, atom_fdim:].T
    w_h_t = W_h.T
    w_o_a_t = W_o_w[:, :atom_fdim].T
    w_o_m_t = W_o_w[:, atom_fdim:].T
    bias = W_o_b.reshape(1, H)

    a_proj = _matmul(atom_features, w_i_a_t, block=1000)
    g_rows = _sc_gather(a_proj, src)
    msg_input, msg = _add_relu(g_rows, bond_features, w_i_b_t, block=2000)

    for _ in range(depth - 1):
        accum = _sc_segment_sum(msg, lsrc, ldst, e, wr=9728,
                                ch2=1600, unroll=4)
        msg = _round_matmul(accum, w_h_t, msg_input, block=2000)

    edge_iota = jnp.arange(e, dtype=jnp.int32)
    m = _sc_segment_sum(msg, edge_iota, dst, n, wr=5120)

    return _final_readout(atom_features, m, graph_ids, w_o_a_t, w_o_m_t,
                          bias, n_graphs, block=1000)
